# Initial kernel scaffold; baseline (speedup 1.0000x reference)
import jax
import jax.numpy as jnp
from jax.experimental import pallas as pl


def kernel(numericalized_doc_toks, embedding):
    raise NotImplementedError("write your pallas kernel here")


# speedup vs baseline: 3.5940x; 3.5940x over previous
import functools

import jax
import jax.numpy as jnp
from jax import lax
from jax.experimental import pallas as pl
from jax.experimental.pallas import tpu as pltpu
from jax.experimental.pallas import tpu_sc as plsc

EMBED_DIM = 64
DOC_LEN = 50
LANES = 16
NCOL = EMBED_DIM // LANES

NC, NS = 2, 16
NW = NC * NS
NBUF = 4


def _pooled_gather_body(idx_hbm, table_hbm, out_hbm, idx_v, rows_v, out_v,
                        s0, s1, s2, s3):
    sems = (s0, s1, s2, s3)
    segs_per_w = idx_v.shape[0]
    wid = lax.axis_index("s") * NC + lax.axis_index("c")
    base = wid * segs_per_w

    pltpu.sync_copy(idx_hbm.at[pl.ds(base, segs_per_w)], idx_v)

    for b in range(NBUF):
        pltpu.async_copy(table_hbm.at[idx_v.at[b]], rows_v.at[b], sems[b])

    scale = jnp.float32(1.0 / DOC_LEN)

    @pl.loop(0, segs_per_w, step=NBUF)
    def _chunk(seg0):
        for b in range(NBUF):
            seg = seg0 + b
            pltpu.make_async_copy(
                table_hbm.at[idx_v.at[seg]], rows_v.at[b], sems[b]).wait()

            def acc_body(l, accs, _b=b):
                return tuple(
                    a + rows_v[_b, l, pl.ds(d * LANES, LANES)]
                    for d, a in enumerate(accs))

            accs = lax.fori_loop(
                0, DOC_LEN, acc_body,
                tuple(jnp.zeros((LANES,), jnp.float32) for _ in range(NCOL)),
                unroll=5)
            for d in range(NCOL):
                out_v[seg, pl.ds(d * LANES, LANES)] = accs[d] * scale

            nxt = seg + NBUF

            @pl.when(nxt < segs_per_w)
            def _prefetch(_b=b, _nxt=nxt):
                pltpu.async_copy(
                    table_hbm.at[idx_v.at[_nxt]], rows_v.at[_b], sems[_b])

    pltpu.sync_copy(out_v, out_hbm.at[pl.ds(base, segs_per_w)])


def kernel(numericalized_doc_toks, embedding):
    batch, n_docs, doc_len = numericalized_doc_toks.shape
    segs = batch * n_docs
    segs_per_w = segs // NW
    idx2d = numericalized_doc_toks.reshape(segs, doc_len)

    mesh = plsc.VectorSubcoreMesh(core_axis_name="c", subcore_axis_name="s")
    run = functools.partial(
        pl.kernel,
        out_type=jax.ShapeDtypeStruct((segs, EMBED_DIM), jnp.float32),
        mesh=mesh,
        scratch_types=[
            pltpu.VMEM((segs_per_w, doc_len), jnp.int32),
            pltpu.VMEM((NBUF, doc_len, EMBED_DIM), jnp.float32),
            pltpu.VMEM((segs_per_w, EMBED_DIM), jnp.float32),
            pltpu.SemaphoreType.DMA,
            pltpu.SemaphoreType.DMA,
            pltpu.SemaphoreType.DMA,
            pltpu.SemaphoreType.DMA,
        ],
        compiler_params=pltpu.CompilerParams(use_tc_tiling_on_sc=False),
    )(_pooled_gather_body)
    out = run(idx2d, embedding)
    return out.reshape(batch, n_docs, EMBED_DIM)


# speedup vs baseline: 3.8560x; 1.0729x over previous
import functools

import jax
import jax.numpy as jnp
from jax import lax
from jax.experimental import pallas as pl
from jax.experimental.pallas import tpu as pltpu
from jax.experimental.pallas import tpu_sc as plsc

EMBED_DIM = 64
DOC_LEN = 50
LANES = 16
NCOL = EMBED_DIM // LANES

NC, NS = 2, 16
NW = NC * NS
PAIR = 2
NBUF = 8


def _pooled_gather_body(idx_hbm, table_hbm, out_hbm, idx_v, rows_v, out_v,
                        *sems):
    chunks_per_w = idx_v.shape[0]
    wid = lax.axis_index("s") * NC + lax.axis_index("c")
    base_c = wid * chunks_per_w

    pltpu.sync_copy(idx_hbm.at[pl.ds(base_c, chunks_per_w)], idx_v)

    for b in range(NBUF):
        pltpu.async_copy(table_hbm.at[idx_v.at[b]], rows_v.at[b], sems[b])

    scale = jnp.float32(1.0 / DOC_LEN)

    @pl.loop(0, chunks_per_w, step=NBUF)
    def _chunk(c0):
        for b in range(NBUF):
            c = c0 + b
            pltpu.make_async_copy(
                table_hbm.at[idx_v.at[c]], rows_v.at[b], sems[b]).wait()

            for p in range(PAIR):
                def acc_body(l, accs, _b=b, _p=p):
                    return tuple(
                        a + rows_v[_b, _p * DOC_LEN + l, pl.ds(d * LANES, LANES)]
                        for d, a in enumerate(accs))

                accs = lax.fori_loop(
                    0, DOC_LEN, acc_body,
                    tuple(jnp.zeros((LANES,), jnp.float32)
                          for _ in range(NCOL)),
                    unroll=10)
                seg = c * PAIR + p
                for d in range(NCOL):
                    out_v[seg, pl.ds(d * LANES, LANES)] = accs[d] * scale

            nxt = c + NBUF

            @pl.when(nxt < chunks_per_w)
            def _prefetch(_b=b, _nxt=nxt):
                pltpu.async_copy(
                    table_hbm.at[idx_v.at[_nxt]], rows_v.at[_b], sems[_b])

    segs_per_w = chunks_per_w * PAIR
    pltpu.sync_copy(out_v, out_hbm.at[pl.ds(wid * segs_per_w, segs_per_w)])


def kernel(numericalized_doc_toks, embedding):
    batch, n_docs, doc_len = numericalized_doc_toks.shape
    segs = batch * n_docs
    segs_per_w = segs // NW
    chunks_per_w = segs_per_w // PAIR
    idx2d = numericalized_doc_toks.reshape(segs // PAIR, doc_len * PAIR)

    mesh = plsc.VectorSubcoreMesh(core_axis_name="c", subcore_axis_name="s")
    run = functools.partial(
        pl.kernel,
        out_type=jax.ShapeDtypeStruct((segs, EMBED_DIM), jnp.float32),
        mesh=mesh,
        scratch_types=[
            pltpu.VMEM((chunks_per_w, doc_len * PAIR), jnp.int32),
            pltpu.VMEM((NBUF, doc_len * PAIR, EMBED_DIM), jnp.float32),
            pltpu.VMEM((segs_per_w, EMBED_DIM), jnp.float32),
        ] + [pltpu.SemaphoreType.DMA] * NBUF,
        compiler_params=pltpu.CompilerParams(use_tc_tiling_on_sc=False),
    )(_pooled_gather_body)
    out = run(idx2d, embedding)
    return out.reshape(batch, n_docs, EMBED_DIM)
